# baseline (speedup 1.0000x reference)
import dataclasses
import functools

import jax
import jax.numpy as jnp
import numpy as np
from jax import lax
from jax.experimental import pallas as pl
from jax.experimental.pallas import tpu as pltpu
from jax.experimental.pallas import tpu_sc as plsc

S = 2048
D = 768
H = 3072
TOPK = 32
K_TOTAL = TOPK * S
N = S * H

NW = 32
NPT = N // NW
CHUNK = 16384
NCHUNK = NPT // CHUNK
NLANE = 16

MIN_I32 = np.int32(-2147483648)


def _f32_key(bits_i32):
    m = lax.shift_right_arithmetic(bits_i32, 31)
    return lax.bitwise_xor(bits_i32, lax.bitwise_or(m, MIN_I32))


def _sc_mesh():
    return plsc.VectorSubcoreMesh(core_axis_name="c", subcore_axis_name="s")


def _sc_params():
    cp = pltpu.CompilerParams()
    if "needs_layout_passes" in pltpu.CompilerParams.__dataclass_fields__:
        cp = dataclasses.replace(cp, needs_layout_passes=False)
    return cp


def _wid():
    return lax.axis_index("s") * 2 + lax.axis_index("c")


UNROLL = 8


def _zero_hist(hist_v, nwords):
    zeros = jnp.zeros((NLANE,), jnp.int32)

    def z(i, _):
        for u in range(UNROLL):
            hist_v[pl.ds((i * UNROLL + u) * NLANE, NLANE)] = zeros
        return 0

    lax.fori_loop(0, nwords // (NLANE * UNROLL), z, 0)


def _hist_words(nb):
    return ((NLANE * (nb + 1) + 127) // 128) * 128


def _reduce_lanes(hist_v, red_v, nb):
    stride = nb + 1

    def red(j, _):
        acc = hist_v[pl.ds(j * NLANE, NLANE)]
        for l in range(1, NLANE):
            acc = acc + hist_v[pl.ds(l * stride + j * NLANE, NLANE)]
        red_v[pl.ds(j * NLANE, NLANE)] = acc
        return 0

    lax.fori_loop(0, nb // NLANE, red, 0)


def _hist_scan(score_refs, out_hbm, buf0_v, buf1_v, hist_v, red_v, sem0, sem1,
               nb, elem_fn):
    wid = _wid()
    _zero_hist(hist_v, _hist_words(nb))
    lanes = lax.iota(jnp.int32, NLANE) * (nb + 1)
    ones = jnp.ones((NLANE,), jnp.int32)
    bufs = [buf0_v, buf1_v]
    sems = [sem0, sem1]
    work = []
    for ref in score_refs:
        npt = ref.shape[0] // NW
        base = wid * npt
        for c in range(npt // CHUNK):
            work.append((ref, base, c))
    handles = [None, None]
    ref0, base0, c0 = work[0]
    handles[0] = pltpu.async_copy(
        ref0.at[pl.ds(base0 + c0 * CHUNK, CHUNK)], bufs[0], sems[0])
    for w in range(len(work)):
        if w + 1 < len(work):
            refn, basen, cn = work[w + 1]
            handles[(w + 1) % 2] = pltpu.async_copy(
                refn.at[pl.ds(basen + cn * CHUNK, CHUNK)],
                bufs[(w + 1) % 2], sems[(w + 1) % 2])
        handles[w % 2].wait()
        buf = bufs[w % 2]

        def inner(i, _):
            b0 = i * (NLANE * UNROLL)
            xs = [buf[pl.ds(b0 + u * NLANE, NLANE)] for u in range(UNROLL)]
            elem_fn(xs, lanes, ones)
            return 0

        lax.fori_loop(0, CHUNK // (NLANE * UNROLL), inner, 0)
    _reduce_lanes(hist_v, red_v, nb)
    pltpu.sync_copy(red_v, out_hbm.at[wid])


_SC_SCRATCH = [
    pltpu.VMEM((CHUNK,), jnp.float32),
    pltpu.VMEM((CHUNK,), jnp.float32),
    pltpu.VMEM((_hist_words(4096),), jnp.int32),
    pltpu.VMEM((4096,), jnp.int32),
    pltpu.SemaphoreType.DMA,
    pltpu.SemaphoreType.DMA,
]


@functools.partial(
    pl.kernel,
    mesh=_sc_mesh(),
    out_type=jax.ShapeDtypeStruct((NW, 4096), jnp.int32),
    scratch_types=_SC_SCRATCH,
    compiler_params=_sc_params(),
)
def _sc_hist1(scores_hbm, out_hbm, buf0_v, buf1_v, hist_v, red_v, sem0, sem1):
    def elem(xs, lanes, ones):
        bits = [plsc.bitcast(x, jnp.int32) for x in xs]
        ms = [lax.shift_right_arithmetic(b, 31) for b in bits]
        sgn = [lax.bitwise_or(m, MIN_I32) for m in ms]
        keys = [lax.bitwise_xor(b, s) for b, s in zip(bits, sgn)]
        idxs = [lanes + lax.shift_right_logical(k, 20) for k in keys]
        for idx in idxs:
            plsc.addupdate_scatter(hist_v, [idx], ones)

    _hist_scan([scores_hbm], out_hbm, buf0_v, buf1_v, hist_v, red_v, sem0,
               sem1, 4096, elem)


@functools.partial(
    pl.kernel,
    mesh=_sc_mesh(),
    out_type=jax.ShapeDtypeStruct((NW, 4096), jnp.int32),
    scratch_types=_SC_SCRATCH + [pltpu.VMEM((NLANE,), jnp.int32)],
    compiler_params=_sc_params(),
)
def _sc_hist2(sa_hbm, sb_hbm, sel1_hbm, out_hbm, buf0_v, buf1_v, hist_v,
              red_v, sem0, sem1, sel_v):
    pltpu.sync_copy(sel1_hbm, sel_v)
    b1 = sel_v[...]
    mask12 = jnp.full((NLANE,), 4095, jnp.int32)

    def elem(xs, lanes, ones):
        bits = [plsc.bitcast(x, jnp.int32) for x in xs]
        ms = [lax.shift_right_arithmetic(b, 31) for b in bits]
        sgn = [lax.bitwise_or(m, MIN_I32) for m in ms]
        keys = [lax.bitwise_xor(b, s) for b, s in zip(bits, sgn)]
        matches = [lax.shift_right_logical(k, 20) == b1 for k in keys]
        idxs = [lanes + lax.bitwise_and(lax.shift_right_logical(k, 8), mask12)
                for k in keys]
        for idx, match in zip(idxs, matches):
            plsc.addupdate_scatter(hist_v, [idx], ones, mask=match)

    _hist_scan([sa_hbm, sb_hbm], out_hbm, buf0_v, buf1_v, hist_v, red_v, sem0,
               sem1, 4096, elem)


@functools.partial(
    pl.kernel,
    mesh=_sc_mesh(),
    out_type=jax.ShapeDtypeStruct((NW, 256), jnp.int32),
    scratch_types=[
        pltpu.VMEM((CHUNK,), jnp.float32),
        pltpu.VMEM((CHUNK,), jnp.float32),
        pltpu.VMEM((_hist_words(256),), jnp.int32),
        pltpu.VMEM((256,), jnp.int32),
        pltpu.SemaphoreType.DMA,
        pltpu.SemaphoreType.DMA,
        pltpu.VMEM((NLANE,), jnp.int32),
        pltpu.VMEM((NLANE,), jnp.int32),
    ],
    compiler_params=_sc_params(),
)
def _sc_hist3(sa_hbm, sb_hbm, sel1_hbm, sel2_hbm, out_hbm, buf0_v, buf1_v,
              hist_v, red_v, sem0, sem1, s1_v, s2_v):
    pltpu.sync_copy(sel1_hbm, s1_v)
    pltpu.sync_copy(sel2_hbm, s2_v)
    prefix = lax.shift_left(s1_v[...], 12) + s2_v[...]
    mask8 = jnp.full((NLANE,), 255, jnp.int32)

    def elem(xs, lanes, ones):
        bits = [plsc.bitcast(x, jnp.int32) for x in xs]
        ms = [lax.shift_right_arithmetic(b, 31) for b in bits]
        sgn = [lax.bitwise_or(m, MIN_I32) for m in ms]
        keys = [lax.bitwise_xor(b, s) for b, s in zip(bits, sgn)]
        matches = [lax.shift_right_logical(k, 8) == prefix for k in keys]
        idxs = [lanes + lax.bitwise_and(k, mask8) for k in keys]
        for idx, match in zip(idxs, matches):
            plsc.addupdate_scatter(hist_v, [idx], ones, mask=match)

    _hist_scan([sa_hbm, sb_hbm], out_hbm, buf0_v, buf1_v, hist_v, red_v, sem0,
               sem1, 256, elem)


def _tri_matmul_exact(piecefn, mat_bf16, n_pieces=3):
    out = None
    for p in range(n_pieces):
        contrib = lax.dot_general(
            piecefn(p), mat_bf16, (((1,), (0,)), ((), ())),
            preferred_element_type=jnp.float32)
        contrib_i = contrib.astype(jnp.int32) << (8 * p)
        out = contrib_i if out is None else out + contrib_i
    return out


def _piece(x_i32, p):
    return ((x_i32 >> (8 * p)) & 0xFF).astype(jnp.bfloat16)


def _suffix_sums(hist3d):
    c = jnp.sum(hist3d, axis=0)
    r = c.shape[0]
    rowsum = jnp.sum(c, axis=1, keepdims=True)
    ii = lax.broadcasted_iota(jnp.int32, (r, r), 0)
    jj = lax.broadcasted_iota(jnp.int32, (r, r), 1)
    wmat = (jj > ii).astype(jnp.bfloat16)
    row_suffix = None
    for p in range(3):
        contrib = lax.dot_general(
            wmat, _piece(rowsum, p), (((1,), (0,)), ((), ())),
            preferred_element_type=jnp.float32)
        contrib_i = contrib.astype(jnp.int32) << (8 * p)
        row_suffix = contrib_i if row_suffix is None else row_suffix + contrib_i
    ii2 = lax.broadcasted_iota(jnp.int32, (128, 128), 0)
    jj2 = lax.broadcasted_iota(jnp.int32, (128, 128), 1)
    umat = (ii2 >= jj2).astype(jnp.bfloat16)
    cs = _tri_matmul_exact(lambda p: _piece(c, p), umat)
    return cs + row_suffix, c


def _select_body(hist_ref, kk_ref, selb_ref, krem_ref):
    kk = jnp.max(kk_ref[...])
    suffix, c = _suffix_sums(hist_ref[...])
    ge = (suffix >= kk).astype(jnp.int32)
    bsel = jnp.sum(ge) - 1
    gcount = jnp.sum(jnp.where(suffix < kk, c, 0))
    krem = kk - gcount
    selb_ref[...] = jnp.zeros((1, 128), jnp.int32) + bsel
    krem_ref[...] = jnp.zeros((1, 128), jnp.int32) + krem


def _make_select(nb):
    r = nb // 128
    return pl.pallas_call(
        _select_body,
        out_shape=(
            jax.ShapeDtypeStruct((1, 128), jnp.int32),
            jax.ShapeDtypeStruct((1, 128), jnp.int32),
        ),
    )


_select4096 = _make_select(4096)


def _select3_body(hist_ref, kk_ref, s1_ref, s2_ref, colmax_ref, mask_ref):
    kk = jnp.max(kk_ref[...])
    suffix, c = _suffix_sums(hist_ref[...])
    ge = (suffix >= kk).astype(jnp.int32)
    b3 = jnp.sum(ge) - 1
    b1 = jnp.max(s1_ref[...])
    b2 = jnp.max(s2_ref[...])
    kth = lax.bitwise_or(
        lax.bitwise_or(lax.shift_left(b1, 20), lax.shift_left(b2, 8)), b3)
    colmax = jnp.max(colmax_ref[...], axis=0, keepdims=True)
    cm_bits = lax.bitcast_convert_type(colmax, jnp.int32)
    cm_key = _f32_key(cm_bits)
    cond = lax.bitwise_xor(cm_key, MIN_I32) >= lax.bitwise_xor(kth, MIN_I32)
    mask_ref[...] = jnp.where(cond, 1.0, 0.0).astype(jnp.float32)


_select3 = pl.pallas_call(
    _select3_body,
    out_shape=jax.ShapeDtypeStruct((1, H), jnp.float32),
)


TH = 512
NHT = H // TH


def _g1_body(x_ref, w1_ref, b1_ref, g_ref):
    acc = lax.dot_general(
        x_ref[...], w1_ref[...], (((1,), (1,)), ((), ())),
        preferred_element_type=jnp.float32)
    g_ref[...] = jnp.maximum(acc + b1_ref[...], 0.0)


_g1_call = pl.pallas_call(
    _g1_body,
    grid=(NHT,),
    in_specs=[
        pl.BlockSpec((S, D), lambda j: (0, 0)),
        pl.BlockSpec((TH, D), lambda j: (j, 0)),
        pl.BlockSpec((1, TH), lambda j: (0, j)),
    ],
    out_specs=pl.BlockSpec((S, TH), lambda j: (0, j)),
    out_shape=jax.ShapeDtypeStruct((S, H), jnp.float32),
)


def _g2_body(g_ref, w2_ref, b2_ref, s_ref, cmax_ref):
    acc = lax.dot_general(
        g_ref[...], w2_ref[...], (((1,), (1,)), ((), ())),
        preferred_element_type=jnp.float32)
    sb = acc + b2_ref[...]
    s_ref[...] = sb
    cmax_ref[...] = jnp.max(sb, axis=0, keepdims=True)


SH = S // 2


def _make_g2(half):
    return pl.pallas_call(
        _g2_body,
        grid=(NHT,),
        in_specs=[
            pl.BlockSpec((SH, H), lambda j: (half, 0)),
            pl.BlockSpec((TH, H), lambda j: (j, 0)),
            pl.BlockSpec((1, TH), lambda j: (0, j)),
        ],
        out_specs=(
            pl.BlockSpec((SH, TH), lambda j: (0, j)),
            pl.BlockSpec((1, TH), lambda j: (0, j)),
        ),
        out_shape=(
            jax.ShapeDtypeStruct((SH, H), jnp.float32),
            jax.ShapeDtypeStruct((1, H), jnp.float32),
        ),
    )


_g2_a = _make_g2(0)
_g2_b = _make_g2(1)

_SQRT_2_OVER_PI = 0.7978845608028654


def _main_body(x_ref, up_ref, mod_ref, upb_ref, modb_ref, maskc_ref,
               maskr_ref, down_ref, downb_ref, out_ref):
    j = pl.program_id(0)
    w_eff = jnp.where(maskc_ref[...] > 0.5, mod_ref[...], up_ref[...])
    b_eff = jnp.where(maskr_ref[...] > 0.5, modb_ref[...], upb_ref[...])
    acc = lax.dot_general(
        x_ref[...], w_eff, (((1,), (1,)), ((), ())),
        preferred_element_type=jnp.float32)
    hpre = acc + b_eff
    inner = _SQRT_2_OVER_PI * (hpre + 0.044715 * (hpre * hpre * hpre))
    hact = 0.5 * hpre * (1.0 + jnp.tanh(inner))
    contrib = lax.dot_general(
        hact, down_ref[...], (((1,), (1,)), ((), ())),
        preferred_element_type=jnp.float32)

    @pl.when(j == 0)
    def _():
        out_ref[...] = contrib + downb_ref[...]

    @pl.when(j != 0)
    def _():
        out_ref[...] = out_ref[...] + contrib


_main_call = pl.pallas_call(
    _main_body,
    grid=(NHT,),
    in_specs=[
        pl.BlockSpec((S, D), lambda j: (0, 0)),
        pl.BlockSpec((TH, D), lambda j: (j, 0)),
        pl.BlockSpec((TH, D), lambda j: (j, 0)),
        pl.BlockSpec((1, TH), lambda j: (0, j)),
        pl.BlockSpec((1, TH), lambda j: (0, j)),
        pl.BlockSpec((TH, 1), lambda j: (j, 0)),
        pl.BlockSpec((1, TH), lambda j: (0, j)),
        pl.BlockSpec((D, TH), lambda j: (0, j)),
        pl.BlockSpec((1, D), lambda j: (0, 0)),
    ],
    out_specs=pl.BlockSpec((S, D), lambda j: (0, 0)),
    out_shape=jax.ShapeDtypeStruct((S, D), jnp.float32),
)


def kernel(inputs, up_w, up_b, gate_w1, gate_b1, gate_w2, gate_b2, mod_w,
           mod_b, down_w, down_b):
    x = inputs.reshape(S, D)
    g = _g1_call(x, gate_w1, gate_b1.reshape(1, H))
    b2r = gate_b2.reshape(1, H)
    scores_a, cmax_a = _g2_a(g, gate_w2, b2r)
    scores_b, cmax_b = _g2_b(g, gate_w2, b2r)
    sa = scores_a.reshape(SH * H)
    sb = scores_b.reshape(SH * H)
    colmax2 = jnp.concatenate([cmax_a, cmax_b], axis=0)

    k0 = jnp.full((1, 128), K_TOTAL, jnp.int32)
    hist1a = _sc_hist1(sa)
    hist1b = _sc_hist1(sb)
    hist1 = jnp.concatenate([hist1a, hist1b], axis=0)
    sel1, krem1 = _select4096(hist1.reshape(2 * NW, 32, 128), k0)
    hist2 = _sc_hist2(sa, sb, sel1[0, :NLANE])
    sel2, krem2 = _select4096(hist2.reshape(NW, 32, 128), krem1)
    hist3 = _sc_hist3(sa, sb, sel1[0, :NLANE], sel2[0, :NLANE])
    mask = _select3(hist3.reshape(NW, 2, 128), krem2, sel1, sel2, colmax2)

    out = _main_call(
        x, up_w, mod_w, up_b.reshape(1, H), mod_b.reshape(1, H),
        mask.reshape(H, 1), mask, down_w, down_b.reshape(1, D))
    return out.reshape(1, S, D)


# speedup vs baseline: 1.0711x; 1.0711x over previous
import dataclasses
import functools

import jax
import jax.numpy as jnp
import numpy as np
from jax import lax
from jax.experimental import pallas as pl
from jax.experimental.pallas import tpu as pltpu
from jax.experimental.pallas import tpu_sc as plsc

S = 2048
D = 768
H = 3072
TOPK = 32
K_TOTAL = TOPK * S
N = S * H

NW = 32
NPT = N // NW
CHUNK = 16384
NCHUNK = NPT // CHUNK
NLANE = 16

MIN_I32 = np.int32(-2147483648)


def _f32_key(bits_i32):
    m = lax.shift_right_arithmetic(bits_i32, 31)
    return lax.bitwise_xor(bits_i32, lax.bitwise_or(m, MIN_I32))


def _sc_mesh():
    return plsc.VectorSubcoreMesh(core_axis_name="c", subcore_axis_name="s")


def _sc_params():
    cp = pltpu.CompilerParams()
    if "needs_layout_passes" in pltpu.CompilerParams.__dataclass_fields__:
        cp = dataclasses.replace(cp, needs_layout_passes=False)
    return cp


def _wid():
    return lax.axis_index("s") * 2 + lax.axis_index("c")


UNROLL = 8


def _zero_hist(hist_v, nwords):
    zeros = jnp.zeros((NLANE,), jnp.int32)

    def z(i, _):
        for u in range(UNROLL):
            hist_v[pl.ds((i * UNROLL + u) * NLANE, NLANE)] = zeros
        return 0

    lax.fori_loop(0, nwords // (NLANE * UNROLL), z, 0)


def _hist_words(nb):
    return ((NLANE * (nb + 1) + 127) // 128) * 128


def _reduce_lanes(hist_v, red_v, nb):
    stride = nb + 1

    def red(j, _):
        acc = hist_v[pl.ds(j * NLANE, NLANE)]
        for l in range(1, NLANE):
            acc = acc + hist_v[pl.ds(l * stride + j * NLANE, NLANE)]
        red_v[pl.ds(j * NLANE, NLANE)] = acc
        return 0

    lax.fori_loop(0, nb // NLANE, red, 0)


def _hist_scan(scores_hbm, out_hbm, buf0_v, buf1_v, hist_v, red_v, sem0, sem1,
               nb, elem_fn):
    wid = _wid()
    base = wid * NPT
    _zero_hist(hist_v, _hist_words(nb))
    lanes = lax.iota(jnp.int32, NLANE) * (nb + 1)
    ones = jnp.ones((NLANE,), jnp.int32)
    bufs = [buf0_v, buf1_v]
    sems = [sem0, sem1]
    handles = [None, None]
    handles[0] = pltpu.async_copy(
        scores_hbm.at[pl.ds(base, CHUNK)], bufs[0], sems[0])
    for c in range(NCHUNK):
        if c + 1 < NCHUNK:
            handles[(c + 1) % 2] = pltpu.async_copy(
                scores_hbm.at[pl.ds(base + (c + 1) * CHUNK, CHUNK)],
                bufs[(c + 1) % 2], sems[(c + 1) % 2])
        handles[c % 2].wait()
        buf = bufs[c % 2]

        def inner(i, _):
            b0 = i * (NLANE * UNROLL)
            xs = [buf[pl.ds(b0 + u * NLANE, NLANE)] for u in range(UNROLL)]
            elem_fn(xs, lanes, ones)
            return 0

        lax.fori_loop(0, CHUNK // (NLANE * UNROLL), inner, 0)
    _reduce_lanes(hist_v, red_v, nb)
    pltpu.sync_copy(red_v, out_hbm.at[wid])


_SC_SCRATCH = [
    pltpu.VMEM((CHUNK,), jnp.float32),
    pltpu.VMEM((CHUNK,), jnp.float32),
    pltpu.VMEM((_hist_words(4096),), jnp.int32),
    pltpu.VMEM((4096,), jnp.int32),
    pltpu.SemaphoreType.DMA,
    pltpu.SemaphoreType.DMA,
]


@functools.partial(
    pl.kernel,
    mesh=_sc_mesh(),
    out_type=jax.ShapeDtypeStruct((NW, 4096), jnp.int32),
    scratch_types=_SC_SCRATCH,
    compiler_params=_sc_params(),
)
def _sc_hist1(scores_hbm, out_hbm, buf0_v, buf1_v, hist_v, red_v, sem0, sem1):
    def elem(xs, lanes, ones):
        bits = [plsc.bitcast(x, jnp.int32) for x in xs]
        ms = [lax.shift_right_arithmetic(b, 31) for b in bits]
        sgn = [lax.bitwise_or(m, MIN_I32) for m in ms]
        keys = [lax.bitwise_xor(b, s) for b, s in zip(bits, sgn)]
        idxs = [lanes + lax.shift_right_logical(k, 20) for k in keys]
        for idx in idxs:
            plsc.addupdate_scatter(hist_v, [idx], ones)

    _hist_scan(scores_hbm, out_hbm, buf0_v, buf1_v, hist_v, red_v, sem0, sem1,
               4096, elem)


@functools.partial(
    pl.kernel,
    mesh=_sc_mesh(),
    out_type=jax.ShapeDtypeStruct((NW, 4096), jnp.int32),
    scratch_types=_SC_SCRATCH + [pltpu.VMEM((NLANE,), jnp.int32)],
    compiler_params=_sc_params(),
)
def _sc_hist2(scores_hbm, sel1_hbm, out_hbm, buf0_v, buf1_v, hist_v, red_v,
              sem0, sem1, sel_v):
    pltpu.sync_copy(sel1_hbm, sel_v)
    b1 = sel_v[...]
    mask12 = jnp.full((NLANE,), 4095, jnp.int32)

    def elem(xs, lanes, ones):
        bits = [plsc.bitcast(x, jnp.int32) for x in xs]
        ms = [lax.shift_right_arithmetic(b, 31) for b in bits]
        sgn = [lax.bitwise_or(m, MIN_I32) for m in ms]
        keys = [lax.bitwise_xor(b, s) for b, s in zip(bits, sgn)]
        matches = [lax.shift_right_logical(k, 20) == b1 for k in keys]
        idxs = [lanes + lax.bitwise_and(lax.shift_right_logical(k, 8), mask12)
                for k in keys]
        for idx, match in zip(idxs, matches):
            plsc.addupdate_scatter(hist_v, [idx], ones, mask=match)

    _hist_scan(scores_hbm, out_hbm, buf0_v, buf1_v, hist_v, red_v, sem0, sem1,
               4096, elem)


@functools.partial(
    pl.kernel,
    mesh=_sc_mesh(),
    out_type=jax.ShapeDtypeStruct((NW, 256), jnp.int32),
    scratch_types=[
        pltpu.VMEM((CHUNK,), jnp.float32),
        pltpu.VMEM((CHUNK,), jnp.float32),
        pltpu.VMEM((_hist_words(256),), jnp.int32),
        pltpu.VMEM((256,), jnp.int32),
        pltpu.SemaphoreType.DMA,
        pltpu.SemaphoreType.DMA,
        pltpu.VMEM((NLANE,), jnp.int32),
        pltpu.VMEM((NLANE,), jnp.int32),
    ],
    compiler_params=_sc_params(),
)
def _sc_hist3(scores_hbm, sel1_hbm, sel2_hbm, out_hbm, buf0_v, buf1_v, hist_v,
              red_v, sem0, sem1, s1_v, s2_v):
    pltpu.sync_copy(sel1_hbm, s1_v)
    pltpu.sync_copy(sel2_hbm, s2_v)
    prefix = lax.shift_left(s1_v[...], 12) + s2_v[...]
    mask8 = jnp.full((NLANE,), 255, jnp.int32)

    def elem(xs, lanes, ones):
        bits = [plsc.bitcast(x, jnp.int32) for x in xs]
        ms = [lax.shift_right_arithmetic(b, 31) for b in bits]
        sgn = [lax.bitwise_or(m, MIN_I32) for m in ms]
        keys = [lax.bitwise_xor(b, s) for b, s in zip(bits, sgn)]
        matches = [lax.shift_right_logical(k, 8) == prefix for k in keys]
        idxs = [lanes + lax.bitwise_and(k, mask8) for k in keys]
        for idx, match in zip(idxs, matches):
            plsc.addupdate_scatter(hist_v, [idx], ones, mask=match)

    _hist_scan(scores_hbm, out_hbm, buf0_v, buf1_v, hist_v, red_v, sem0, sem1,
               256, elem)


def _tri_matmul_exact(piecefn, mat_bf16, n_pieces=3):
    out = None
    for p in range(n_pieces):
        contrib = lax.dot_general(
            piecefn(p), mat_bf16, (((1,), (0,)), ((), ())),
            preferred_element_type=jnp.float32)
        contrib_i = contrib.astype(jnp.int32) << (8 * p)
        out = contrib_i if out is None else out + contrib_i
    return out


def _piece(x_i32, p):
    return ((x_i32 >> (8 * p)) & 0xFF).astype(jnp.bfloat16)


def _suffix_sums(hist3d):
    c = jnp.sum(hist3d, axis=0)
    r = c.shape[0]
    rowsum = jnp.sum(c, axis=1, keepdims=True)
    ii = lax.broadcasted_iota(jnp.int32, (r, r), 0)
    jj = lax.broadcasted_iota(jnp.int32, (r, r), 1)
    wmat = (jj > ii).astype(jnp.bfloat16)
    row_suffix = None
    for p in range(3):
        contrib = lax.dot_general(
            wmat, _piece(rowsum, p), (((1,), (0,)), ((), ())),
            preferred_element_type=jnp.float32)
        contrib_i = contrib.astype(jnp.int32) << (8 * p)
        row_suffix = contrib_i if row_suffix is None else row_suffix + contrib_i
    ii2 = lax.broadcasted_iota(jnp.int32, (128, 128), 0)
    jj2 = lax.broadcasted_iota(jnp.int32, (128, 128), 1)
    umat = (ii2 >= jj2).astype(jnp.bfloat16)
    cs = _tri_matmul_exact(lambda p: _piece(c, p), umat)
    return cs + row_suffix, c


def _select_body(hist_ref, kk_ref, selb_ref, krem_ref):
    kk = jnp.max(kk_ref[...])
    suffix, c = _suffix_sums(hist_ref[...])
    ge = (suffix >= kk).astype(jnp.int32)
    bsel = jnp.sum(ge) - 1
    gcount = jnp.sum(jnp.where(suffix < kk, c, 0))
    krem = kk - gcount
    selb_ref[...] = jnp.zeros((1, 128), jnp.int32) + bsel
    krem_ref[...] = jnp.zeros((1, 128), jnp.int32) + krem


def _make_select(nb):
    r = nb // 128
    return pl.pallas_call(
        _select_body,
        out_shape=(
            jax.ShapeDtypeStruct((1, 128), jnp.int32),
            jax.ShapeDtypeStruct((1, 128), jnp.int32),
        ),
    )


_select4096 = _make_select(4096)


def _select3_body(hist_ref, kk_ref, s1_ref, s2_ref, colmax_ref, mask_ref):
    kk = jnp.max(kk_ref[...])
    suffix, c = _suffix_sums(hist_ref[...])
    ge = (suffix >= kk).astype(jnp.int32)
    b3 = jnp.sum(ge) - 1
    b1 = jnp.max(s1_ref[...])
    b2 = jnp.max(s2_ref[...])
    kth = lax.bitwise_or(
        lax.bitwise_or(lax.shift_left(b1, 20), lax.shift_left(b2, 8)), b3)
    cm_bits = lax.bitcast_convert_type(colmax_ref[...], jnp.int32)
    cm_key = _f32_key(cm_bits)
    cond = lax.bitwise_xor(cm_key, MIN_I32) >= lax.bitwise_xor(kth, MIN_I32)
    mask_ref[...] = jnp.where(cond, 1.0, 0.0).astype(jnp.float32)


_select3 = pl.pallas_call(
    _select3_body,
    out_shape=jax.ShapeDtypeStruct((1, H), jnp.float32),
)


TH = 512
NHT = H // TH


def _g1_body(x_ref, w1_ref, b1_ref, g_ref):
    acc = lax.dot_general(
        x_ref[...], w1_ref[...], (((1,), (1,)), ((), ())),
        preferred_element_type=jnp.float32)
    g_ref[...] = jnp.maximum(acc + b1_ref[...], 0.0)


_g1_call = pl.pallas_call(
    _g1_body,
    grid=(NHT,),
    in_specs=[
        pl.BlockSpec((S, D), lambda j: (0, 0)),
        pl.BlockSpec((TH, D), lambda j: (j, 0)),
        pl.BlockSpec((1, TH), lambda j: (0, j)),
    ],
    out_specs=pl.BlockSpec((S, TH), lambda j: (0, j)),
    out_shape=jax.ShapeDtypeStruct((S, H), jnp.float32),
)


def _g2_body(g_ref, w2_ref, b2_ref, s_ref, cmax_ref):
    acc = lax.dot_general(
        g_ref[...], w2_ref[...], (((1,), (1,)), ((), ())),
        preferred_element_type=jnp.float32)
    sb = acc + b2_ref[...]
    s_ref[...] = sb
    cmax_ref[...] = jnp.max(sb, axis=0, keepdims=True)


_g2_call = pl.pallas_call(
    _g2_body,
    grid=(NHT,),
    in_specs=[
        pl.BlockSpec((S, H), lambda j: (0, 0)),
        pl.BlockSpec((TH, H), lambda j: (j, 0)),
        pl.BlockSpec((1, TH), lambda j: (0, j)),
    ],
    out_specs=(
        pl.BlockSpec((S, TH), lambda j: (0, j)),
        pl.BlockSpec((1, TH), lambda j: (0, j)),
    ),
    out_shape=(
        jax.ShapeDtypeStruct((S, H), jnp.float32),
        jax.ShapeDtypeStruct((1, H), jnp.float32),
    ),
)

_SQRT_2_OVER_PI = 0.7978845608028654


def _main_body(x_ref, up_ref, mod_ref, upb_ref, modb_ref, maskc_ref,
               maskr_ref, down_ref, downb_ref, out_ref):
    j = pl.program_id(0)
    w_eff = jnp.where(maskc_ref[...] > 0.5, mod_ref[...], up_ref[...])
    b_eff = jnp.where(maskr_ref[...] > 0.5, modb_ref[...], upb_ref[...])
    acc = lax.dot_general(
        x_ref[...], w_eff, (((1,), (1,)), ((), ())),
        preferred_element_type=jnp.float32)
    hpre = acc + b_eff
    inner = _SQRT_2_OVER_PI * (hpre + 0.044715 * (hpre * hpre * hpre))
    hact = 0.5 * hpre * (1.0 + jnp.tanh(inner))
    contrib = lax.dot_general(
        hact, down_ref[...], (((1,), (1,)), ((), ())),
        preferred_element_type=jnp.float32)

    @pl.when(j == 0)
    def _():
        out_ref[...] = contrib + downb_ref[...]

    @pl.when(j != 0)
    def _():
        out_ref[...] = out_ref[...] + contrib


_main_call = pl.pallas_call(
    _main_body,
    grid=(NHT,),
    in_specs=[
        pl.BlockSpec((S, D), lambda j: (0, 0)),
        pl.BlockSpec((TH, D), lambda j: (j, 0)),
        pl.BlockSpec((TH, D), lambda j: (j, 0)),
        pl.BlockSpec((1, TH), lambda j: (0, j)),
        pl.BlockSpec((1, TH), lambda j: (0, j)),
        pl.BlockSpec((TH, 1), lambda j: (j, 0)),
        pl.BlockSpec((1, TH), lambda j: (0, j)),
        pl.BlockSpec((D, TH), lambda j: (0, j)),
        pl.BlockSpec((1, D), lambda j: (0, 0)),
    ],
    out_specs=pl.BlockSpec((S, D), lambda j: (0, 0)),
    out_shape=jax.ShapeDtypeStruct((S, D), jnp.float32),
)


def kernel(inputs, up_w, up_b, gate_w1, gate_b1, gate_w2, gate_b2, mod_w,
           mod_b, down_w, down_b):
    x = inputs.reshape(S, D)
    g = _g1_call(x, gate_w1, gate_b1.reshape(1, H))
    scores, colmax = _g2_call(g, gate_w2, gate_b2.reshape(1, H))
    scores_flat = scores.reshape(N)

    k0 = jnp.full((1, 128), K_TOTAL, jnp.int32)
    hist1 = _sc_hist1(scores_flat)
    sel1, krem1 = _select4096(hist1.reshape(NW, 32, 128), k0)
    hist2 = _sc_hist2(scores_flat, sel1[0, :NLANE])
    sel2, krem2 = _select4096(hist2.reshape(NW, 32, 128), krem1)
    hist3 = _sc_hist3(scores_flat, sel1[0, :NLANE], sel2[0, :NLANE])
    mask = _select3(hist3.reshape(NW, 2, 128), krem2, sel1, sel2, colmax)

    out = _main_call(
        x, up_w, mod_w, up_b.reshape(1, H), mod_b.reshape(1, H),
        mask.reshape(H, 1), mask, down_w, down_b.reshape(1, D))
    return out.reshape(1, S, D)
